# baseline (speedup 1.0000x reference)
import functools

import jax
import jax.numpy as jnp
from jax import lax
from jax.experimental import pallas as pl
from jax.experimental.pallas import tpu as pltpu
from jax.experimental.pallas import tpu_sc as plsc

_BT = 512
_K = 8192
_D = 256
_N = 16384


def _l2e_body(e_ref, l2e_ref):
    e0 = e_ref[...]
    l2e_ref[...] = jnp.sum(e0 * e0, axis=1)[None, :]


def _l2e_pallas(embeddings):
    return pl.pallas_call(
        _l2e_body,
        out_shape=jax.ShapeDtypeStruct((1, _K), jnp.float32),
    )(embeddings)


def _argmin_body(x_ref, e_ref, l2e_ref, codes_ref):
    x = x_ref[...]
    e = e_ref[...]
    dot2 = lax.dot_general(x + x, e, (((1,), (1,)), ((), ())),
                           preferred_element_type=jnp.float32)
    l2x = jnp.sum(x * x, axis=1, keepdims=True)
    dist = (l2x + l2e_ref[...]) - dot2
    codes_ref[0, 0, :] = jnp.argmin(dist, axis=1).astype(jnp.int32)


def _codes_half(x, embeddings, l2e, block_off, nb):
    codes3 = pl.pallas_call(
        _argmin_body,
        grid=(nb,),
        in_specs=[
            pl.BlockSpec((_BT, _D), lambda i: (i + block_off, 0)),
            pl.BlockSpec((_K, _D), lambda i: (0, 0)),
            pl.BlockSpec((1, _K), lambda i: (0, 0)),
        ],
        out_specs=pl.BlockSpec((1, 1, _BT), lambda i: (i, 0, 0)),
        out_shape=jax.ShapeDtypeStruct((nb, 1, _BT), jnp.int32),
    )(x, embeddings, l2e)
    return codes3.reshape(nb * _BT)


def _gather_sc(embeddings, codes_flat):
    info = plsc.get_sparse_core_info()
    nw = info.num_cores * info.num_subcores
    b = codes_flat.shape[0]
    b_per_w = b // nw
    ch = 128
    nch = b_per_w // ch
    mesh = plsc.VectorSubcoreMesh(core_axis_name="c", subcore_axis_name="s")

    @functools.partial(
        pl.kernel, mesh=mesh,
        out_type=jax.ShapeDtypeStruct((b, _D), jnp.float32),
        scratch_types=[
            pltpu.VMEM((b_per_w,), jnp.int32),
            pltpu.VMEM((ch, _D), jnp.float32),
            pltpu.VMEM((ch, _D), jnp.float32),
            pltpu.SemaphoreType.DMA,
            pltpu.SemaphoreType.DMA,
        ],
    )
    def k(table_hbm, idx_hbm, out_hbm, idx_v, rows0, rows1, sem0, sem1):
        wid = lax.axis_index("s") * info.num_cores + lax.axis_index("c")
        base = wid * b_per_w
        rows = (rows0, rows1)
        sems = (sem0, sem1)
        pltpu.sync_copy(idx_hbm.at[pl.ds(base, b_per_w)], idx_v)
        cps = [None, None]
        cps[0] = pltpu.async_copy(
            table_hbm.at[idx_v.at[pl.ds(0, ch)]], rows[0], sems[0])
        for c in range(nch):
            nxt = c + 1
            if nxt < nch:
                cps[nxt % 2] = pltpu.async_copy(
                    table_hbm.at[idx_v.at[pl.ds(nxt * ch, ch)]],
                    rows[nxt % 2], sems[nxt % 2])
            cps[c % 2].wait()
            pltpu.sync_copy(rows[c % 2], out_hbm.at[pl.ds(base + c * ch, ch)])

    return k(embeddings, codes_flat)


def kernel(inputs, embeddings):
    bsz, h, w, d = inputs.shape
    n = bsz * h * w
    x = inputs.reshape(n, d)
    l2e = _l2e_pallas(embeddings)
    nb = n // _BT
    half = nb // 2
    codes0 = _codes_half(x, embeddings, l2e, 0, half)
    codes1 = _codes_half(x, embeddings, l2e, half, nb - half)
    vecs0 = _gather_sc(embeddings, codes0)
    vecs1 = _gather_sc(embeddings, codes1)
    codes_flat = jnp.concatenate([codes0, codes1])
    vecs = jnp.concatenate([vecs0, vecs1], axis=0)
    return codes_flat.reshape(bsz, h, w), vecs.reshape(bsz, h, w, d)


# speedup vs baseline: 1.0919x; 1.0919x over previous
import functools

import jax
import jax.numpy as jnp
from jax import lax
from jax.experimental import pallas as pl
from jax.experimental.pallas import tpu as pltpu
from jax.experimental.pallas import tpu_sc as plsc

_BT = 512
_K = 8192
_D = 256


def _argmin_body(x_ref, e_ref, codes_ref, l2e_ref):
    @pl.when(pl.program_id(0) == 0)
    def _():
        e0 = e_ref[...]
        l2e_ref[...] = jnp.sum(e0 * e0, axis=1)[None, :]

    x = x_ref[...]
    e = e_ref[...]
    dot2 = lax.dot_general(x + x, e, (((1,), (1,)), ((), ())),
                           preferred_element_type=jnp.float32)
    l2x = jnp.sum(x * x, axis=1, keepdims=True)
    l2e = l2e_ref[...]
    dist = (l2x + l2e) - dot2
    codes_ref[0, 0, :] = jnp.argmin(dist, axis=1).astype(jnp.int32)


def _codes_pallas(x, embeddings):
    n = x.shape[0]
    nb = n // _BT
    codes3 = pl.pallas_call(
        _argmin_body,
        grid=(nb,),
        in_specs=[
            pl.BlockSpec((_BT, _D), lambda i: (i, 0)),
            pl.BlockSpec((_K, _D), lambda i: (0, 0)),
        ],
        out_specs=pl.BlockSpec((1, 1, _BT), lambda i: (i, 0, 0)),
        out_shape=jax.ShapeDtypeStruct((nb, 1, _BT), jnp.int32),
        scratch_shapes=[pltpu.VMEM((1, _K), jnp.float32)],
    )(x, embeddings)
    return codes3.reshape(n)


def _gather_sc(embeddings, codes_flat):
    info = plsc.get_sparse_core_info()
    nw = info.num_cores * info.num_subcores
    b = codes_flat.shape[0]
    b_per_w = b // nw
    ch = 128
    nch = b_per_w // ch
    mesh = plsc.VectorSubcoreMesh(core_axis_name="c", subcore_axis_name="s")

    @functools.partial(
        pl.kernel, mesh=mesh,
        out_type=jax.ShapeDtypeStruct((b, _D), jnp.float32),
        scratch_types=[
            pltpu.VMEM((b_per_w,), jnp.int32),
            pltpu.VMEM((ch, _D), jnp.float32),
            pltpu.VMEM((ch, _D), jnp.float32),
            pltpu.SemaphoreType.DMA,
            pltpu.SemaphoreType.DMA,
        ],
    )
    def k(table_hbm, idx_hbm, out_hbm, idx_v, rows0, rows1, sem0, sem1):
        wid = lax.axis_index("s") * info.num_cores + lax.axis_index("c")
        base = wid * b_per_w
        rows = (rows0, rows1)
        sems = (sem0, sem1)
        pltpu.sync_copy(idx_hbm.at[pl.ds(base, b_per_w)], idx_v)
        cps = [None, None]
        cps[0] = pltpu.async_copy(
            table_hbm.at[idx_v.at[pl.ds(0, ch)]], rows[0], sems[0])
        for c in range(nch):
            nxt = c + 1
            if nxt < nch:
                cps[nxt % 2] = pltpu.async_copy(
                    table_hbm.at[idx_v.at[pl.ds(nxt * ch, ch)]],
                    rows[nxt % 2], sems[nxt % 2])
            cps[c % 2].wait()
            pltpu.sync_copy(rows[c % 2], out_hbm.at[pl.ds(base + c * ch, ch)])

    return k(embeddings, codes_flat)


def kernel(inputs, embeddings):
    bsz, h, w, d = inputs.shape
    n = bsz * h * w
    x = inputs.reshape(n, d)
    codes_flat = _codes_pallas(x, embeddings)
    code_vecs = _gather_sc(embeddings, codes_flat)
    return codes_flat.reshape(bsz, h, w), code_vecs.reshape(bsz, h, w, d)
